# baseline (speedup 1.0000x reference)
import functools

import jax
import jax.numpy as jnp
from jax import lax
from jax.experimental import pallas as pl
from jax.experimental.pallas import tpu as pltpu
from jax.experimental.pallas import tpu_sc as plsc

_S = 2
_R = 16
_NBUF = 7


def kernel(x):
    B, N, D = x.shape
    Dp = D // _S
    M = B * N * _S
    xf = x.reshape(M, Dp)
    NW = 32
    rows_per_w = M // NW
    n_chunks = rows_per_w // _R
    mesh = plsc.VectorSubcoreMesh(core_axis_name="c", subcore_axis_name="s")

    @functools.partial(
        pl.kernel,
        mesh=mesh,
        out_type=jax.ShapeDtypeStruct((M, Dp), jnp.float32),
        scratch_types=[
            pltpu.VMEM((_NBUF, _R), jnp.int32),
            pltpu.VMEM((_NBUF, _R, Dp), jnp.float32),
        ]
        + [pltpu.SemaphoreType.DMA] * (2 * _NBUF),
    )
    def k(x_hbm, out_hbm, idx_v, rows_v, *sems):
        gsems, wsems = sems[:_NBUF], sems[_NBUF:]
        wid = lax.axis_index("s") * 2 + lax.axis_index("c")
        base = wid * rows_per_w

        def start_gather(t, slot):
            rbase = base + t * _R
            b = rbase // (N * _S)
            shift = _S.bit_length() - 1
            rv = jnp.full((_R,), rbase, jnp.int32) + lax.iota(jnp.int32, _R)
            kv = lax.shift_right_logical(rv, shift)
            hv = rv & (_S - 1)
            idx_v[slot, :] = _S * (2 * b * N + N - 1) - lax.shift_left(kv, shift) + hv
            pltpu.async_copy(x_hbm.at[idx_v.at[slot]], rows_v.at[slot], gsems[slot])

        def wait_gather(slot):
            pltpu.make_async_copy(
                x_hbm.at[idx_v.at[slot]], rows_v.at[slot], gsems[slot]
            ).wait()

        def start_write(t, slot):
            pltpu.async_copy(
                rows_v.at[slot], out_hbm.at[pl.ds(base + t * _R, _R)], wsems[slot]
            )

        def wait_write(slot):
            pltpu.make_async_copy(
                rows_v.at[slot], out_hbm.at[pl.ds(base, _R)], wsems[slot]
            ).wait()

        for s in range(_NBUF - 1):
            start_gather(s, s)

        def main_body(step, _):
            for u in range(_NBUF):
                t = step * _NBUF + u
                slot = u

                @pl.when(t < n_chunks)
                def _():
                    wait_gather(slot)
                    start_write(t, slot)
                    t2 = t + _NBUF - 1
                    slot2 = (u + _NBUF - 1) % _NBUF

                    @pl.when(t2 < n_chunks)
                    def _():
                        @pl.when(t2 >= _NBUF)
                        def _():
                            wait_write(slot2)

                        start_gather(t2, slot2)

            return 0

        nsteps = (n_chunks + _NBUF - 1) // _NBUF
        lax.fori_loop(0, nsteps, main_body, 0)
        for s in range(_NBUF):
            if any(t % _NBUF == s for t in range(max(0, n_chunks - _NBUF), n_chunks)):
                wait_write(s)

    return k(xf).reshape(B, N, D)


# speedup vs baseline: 3.5783x; 3.5783x over previous
import functools

import jax
import jax.numpy as jnp
from jax import lax
from jax.experimental import pallas as pl
from jax.experimental.pallas import tpu as pltpu
from jax.experimental.pallas import tpu_sc as plsc

_R = 16
_NBUF = 3


def kernel(x):
    B, N, D = x.shape
    M = B * N
    xf = x.reshape(M, D)
    NW = 32
    rows_per_w = M // NW
    n_chunks = rows_per_w // _R
    mesh = plsc.VectorSubcoreMesh(core_axis_name="c", subcore_axis_name="s")

    @functools.partial(
        pl.kernel,
        mesh=mesh,
        out_type=jax.ShapeDtypeStruct((M, D), jnp.float32),
        scratch_types=[
            pltpu.VMEM((_NBUF, _R), jnp.int32),
            pltpu.VMEM((_NBUF, _R, D), jnp.float32),
        ]
        + [pltpu.SemaphoreType.DMA] * (2 * _NBUF),
    )
    def k(x_hbm, out_hbm, idx_v, rows_v, *sems):
        gsems, wsems = sems[:_NBUF], sems[_NBUF:]
        wid = lax.axis_index("s") * 2 + lax.axis_index("c")
        base = wid * rows_per_w

        def srclo(t):
            obase = base + t * _R
            b = obase // N
            return 2 * b * N + N - _R - obase

        def start_gather(t, slot):
            pltpu.async_copy(
                x_hbm.at[pl.ds(srclo(t), _R)], rows_v.at[slot], gsems[slot]
            )

        def wait_gather(slot):
            pltpu.make_async_copy(
                x_hbm.at[pl.ds(0, _R)], rows_v.at[slot], gsems[slot]
            ).wait()

        def start_write(t, slot):
            top = base + t * _R + _R - 1
            idx_v[slot, :] = jnp.full((_R,), top, jnp.int32) - lax.iota(
                jnp.int32, _R
            )
            pltpu.async_copy(
                rows_v.at[slot], out_hbm.at[idx_v.at[slot]], wsems[slot]
            )

        def wait_write(slot):
            pltpu.make_async_copy(
                rows_v.at[slot], out_hbm.at[idx_v.at[slot]], wsems[slot]
            ).wait()

        for s in range(_NBUF - 1):
            start_gather(s, s)

        def main_body(step, _):
            for u in range(_NBUF):
                t = step * _NBUF + u
                slot = u

                @pl.when(t < n_chunks)
                def _():
                    wait_gather(slot)
                    start_write(t, slot)
                    t2 = t + _NBUF - 1
                    slot2 = (u + _NBUF - 1) % _NBUF

                    @pl.when(t2 < n_chunks)
                    def _():
                        @pl.when(t2 >= _NBUF)
                        def _():
                            wait_write(slot2)

                        start_gather(t2, slot2)

            return 0

        nsteps = (n_chunks + _NBUF - 1) // _NBUF
        lax.fori_loop(0, nsteps, main_body, 0)
        for s in range(_NBUF):
            if any(t % _NBUF == s for t in range(max(0, n_chunks - _NBUF), n_chunks)):
                wait_write(s)

    return k(xf).reshape(B, N, D)


# speedup vs baseline: 3.6878x; 1.0306x over previous
import functools

import jax
import jax.numpy as jnp
from jax import lax
from jax.experimental import pallas as pl
from jax.experimental.pallas import tpu as pltpu
from jax.experimental.pallas import tpu_sc as plsc

_R = 8
_NBUF = 6


def kernel(x):
    B, N, D = x.shape
    M = B * N
    xf = x.reshape(M, D)
    NW = 32
    rows_per_w = M // NW
    n_chunks = rows_per_w // _R
    mesh = plsc.VectorSubcoreMesh(core_axis_name="c", subcore_axis_name="s")

    @functools.partial(
        pl.kernel,
        mesh=mesh,
        out_type=jax.ShapeDtypeStruct((M, D), jnp.float32),
        scratch_types=[
            pltpu.VMEM((rows_per_w,), jnp.int32),
            pltpu.VMEM((_NBUF, _R, D), jnp.float32),
        ]
        + [pltpu.SemaphoreType.DMA] * (2 * _NBUF),
    )
    def k(x_hbm, out_hbm, idx_flat, rows_v, *sems):
        gsems, wsems = sems[:_NBUF], sems[_NBUF:]
        wid = lax.axis_index("s") * 2 + lax.axis_index("c")
        base = wid * rows_per_w
        src0 = 2 * (base // N) * N + N - 1 - base

        def fill(i, _):
            idx_flat[pl.ds(i * 16, 16)] = (
                jnp.full((16,), src0 - i * 16, jnp.int32) - lax.iota(jnp.int32, 16)
            )
            return 0

        lax.fori_loop(0, rows_per_w // 16, fill, 0)

        def start_gather(t, slot):
            pltpu.async_copy(
                x_hbm.at[idx_flat.at[pl.ds(t * _R, _R)]],
                rows_v.at[slot],
                gsems[slot],
            )

        def wait_gather(slot):
            pltpu.make_async_copy(
                x_hbm.at[idx_flat.at[pl.ds(0, _R)]], rows_v.at[slot], gsems[slot]
            ).wait()

        def start_write(t, slot):
            pltpu.async_copy(
                rows_v.at[slot], out_hbm.at[pl.ds(base + t * _R, _R)], wsems[slot]
            )

        def wait_write(slot):
            pltpu.make_async_copy(
                rows_v.at[slot], out_hbm.at[pl.ds(base, _R)], wsems[slot]
            ).wait()

        for s in range(_NBUF - 1):
            start_gather(s, s)

        def main_body(step, _):
            for u in range(_NBUF):
                t = step * _NBUF + u
                slot = u

                @pl.when(t < n_chunks)
                def _():
                    wait_gather(slot)
                    start_write(t, slot)
                    t2 = t + _NBUF - 1
                    slot2 = (u + _NBUF - 1) % _NBUF

                    @pl.when(t2 < n_chunks)
                    def _():
                        @pl.when(t2 >= _NBUF)
                        def _():
                            wait_write(slot2)

                        start_gather(t2, slot2)

            return 0

        nsteps = (n_chunks + _NBUF - 1) // _NBUF
        lax.fori_loop(0, nsteps, main_body, 0)
        for s in range(_NBUF):
            if any(t % _NBUF == s for t in range(max(0, n_chunks - _NBUF), n_chunks)):
                wait_write(s)

    return k(xf).reshape(B, N, D)
